# baseline (device time: 62606 ns/iter reference)
import jax
import jax.numpy as jnp
from jax import lax
from jax.experimental import pallas as pl
from jax.experimental.pallas import tpu as pltpu

N_RING = 8


def kernel(x):
    m, n = x.shape
    rows = m // N_RING
    hrows = rows // 2

    def body(
        x_hbm,
        out_ref,
        xchunk,
        mysend,
        p1recv,
        copy_sem,
        p1_send, p1_recv,
        cwd_send, cwd_recv,
        cws_send, cws_recv,
        ccw3_send, ccw3_recv,
        ccw2_send, ccw2_recv,
        xf_send, xf_recv,
    ):
        my_x = lax.axis_index("x")
        my_y = lax.axis_index("y")
        my_z = lax.axis_index("z")
        partner = (1 - my_x, my_y, my_z)
        hd = my_x
        hsh = 1 - my_x

        r = jnp.where(my_y == 0, my_z, 7 - my_z)

        def ring_coords(p):
            p = p % N_RING
            py = (p >= 4).astype(my_z.dtype)
            pz = jnp.where(p < 4, p, 7 - p)
            return (my_x, py, pz)

        nxt = ring_coords(r + 1)
        prv = ring_coords(r - 1)

        def hs(k, hf):
            return pl.ds((k % N_RING) * rows + hf * hrows, hrows)

        my_rows = pl.ds(r * rows, rows)
        cp = pltpu.make_async_copy(x_hbm.at[my_rows], xchunk, copy_sem)
        cp.start()

        barrier_sem = pltpu.get_barrier_semaphore()
        for dev in (partner, nxt, prv):
            pl.semaphore_signal(
                barrier_sem, inc=1, device_id=dev,
                device_id_type=pl.DeviceIdType.MESH,
            )
        pl.semaphore_wait(barrier_sem, 3)

        cp.wait()
        mysend[...] = xchunk[...].astype(jnp.bfloat16)

        sends = []

        def start(d):
            d.start()
            sends.append(d)

        def rcopy(slc, send_sems, recv_sems, idx, dev, dst_slc=None):
            return pltpu.make_async_remote_copy(
                src_ref=out_ref.at[slc],
                dst_ref=out_ref.at[dst_slc if dst_slc is not None else slc],
                send_sem=send_sems.at[idx],
                recv_sem=recv_sems.at[idx],
                device_id=dev,
                device_id_type=pl.DeviceIdType.MESH,
            )

        p1 = []
        for i, hf in enumerate((hsh, hd)):
            d = pltpu.make_async_remote_copy(
                src_ref=mysend.at[pl.ds(hf * hrows, hrows)],
                dst_ref=p1recv.at[pl.ds(hf * hrows, hrows)],
                send_sem=p1_send.at[i],
                recv_sem=p1_recv.at[i],
                device_id=partner,
                device_id_type=pl.DeviceIdType.MESH,
            )
            start(d)
            p1.append(d)

        def reduce_half(hf):
            src = pl.ds(hf * hrows, hrows)
            out_ref[hs(r, hf), :] = (
                xchunk[src, :] + p1recv[src, :].astype(jnp.float32)
            ).astype(jnp.bfloat16)

        p1[0].wait_recv()
        reduce_half(hd)
        start(rcopy(hs(r, hd), cwd_send, cwd_recv, 0, nxt))
        start(rcopy(hs(r, hd), ccw2_send, ccw2_recv, 0, prv))
        p1[1].wait_recv()
        reduce_half(hsh)
        start(rcopy(hs(r, hsh), cws_send, cws_recv, 0, nxt))
        start(rcopy(hs(r, hsh), ccw3_send, ccw3_recv, 0, prv))

        for j in range(4):
            rcopy(hs(r - 1 - j, hd), cwd_send, cwd_recv, j, nxt).wait_recv()
            if j + 1 < 4:
                start(rcopy(hs(r - 1 - j, hd), cwd_send, cwd_recv, j + 1, nxt))
            if j == 2:
                start(rcopy(hs(r - 3, hd), xf_send, xf_recv, 0, partner))
            if j == 3:
                start(rcopy(hs(r - 4, hd), xf_send, xf_recv, 1, partner))
            if j < 2:
                rcopy(hs(r - 1 - j, hsh), cws_send, cws_recv, j, nxt).wait_recv()
                if j + 1 < 2:
                    start(
                        rcopy(hs(r - 1 - j, hsh), cws_send, cws_recv, j + 1, nxt)
                    )
            if j < 3:
                rcopy(
                    hs(r + 1 + j, hsh), ccw3_send, ccw3_recv, j, prv
                ).wait_recv()
                if j + 1 < 3:
                    start(
                        rcopy(
                            hs(r + 1 + j, hsh), ccw3_send, ccw3_recv, j + 1, prv
                        )
                    )
                if j == 2:
                    start(rcopy(hs(r + 3, hsh), xf_send, xf_recv, 2, partner))
            if j < 2:
                rcopy(
                    hs(r + 1 + j, hd), ccw2_send, ccw2_recv, j, prv
                ).wait_recv()
                if j + 1 < 2:
                    start(
                        rcopy(hs(r + 1 + j, hd), ccw2_send, ccw2_recv, j + 1, prv)
                    )

        rcopy(hs(r - 3, hsh), xf_send, xf_recv, 0, partner).wait_recv()
        rcopy(hs(r - 4, hsh), xf_send, xf_recv, 1, partner).wait_recv()
        rcopy(hs(r + 3, hd), xf_send, xf_recv, 2, partner).wait_recv()

        for d in sends:
            d.wait_send()

    return pl.pallas_call(
        body,
        out_shape=jax.ShapeDtypeStruct((m, n), jnp.bfloat16),
        in_specs=[pl.BlockSpec(memory_space=pl.ANY)],
        out_specs=pl.BlockSpec(memory_space=pltpu.VMEM),
        scratch_shapes=[
            pltpu.VMEM((rows, n), jnp.float32),
            pltpu.VMEM((rows, n), jnp.bfloat16),
            pltpu.VMEM((rows, n), jnp.bfloat16),
            pltpu.SemaphoreType.DMA,
            pltpu.SemaphoreType.DMA((2,)), pltpu.SemaphoreType.DMA((2,)),
            pltpu.SemaphoreType.DMA((4,)), pltpu.SemaphoreType.DMA((4,)),
            pltpu.SemaphoreType.DMA((2,)), pltpu.SemaphoreType.DMA((2,)),
            pltpu.SemaphoreType.DMA((3,)), pltpu.SemaphoreType.DMA((3,)),
            pltpu.SemaphoreType.DMA((2,)), pltpu.SemaphoreType.DMA((2,)),
            pltpu.SemaphoreType.DMA((3,)), pltpu.SemaphoreType.DMA((3,)),
        ],
        compiler_params=pltpu.CompilerParams(collective_id=0),
    )(x)


# device time: 53417 ns/iter; 1.1720x vs baseline; 1.1720x over previous
import jax
import jax.numpy as jnp
from jax import lax
from jax.experimental import pallas as pl
from jax.experimental.pallas import tpu as pltpu

N_RING = 8
N_Q = 4
CW_DEPTH = (4, 4, 3, 3)
CCW_DEPTH = (3, 3, 4, 4)


def kernel(x):
    m, n = x.shape
    rows = m // N_RING
    qrows = rows // N_Q

    def body(x_hbm, out_ref, xchunk, mysend, p1recv, copy_sem, *sems):
        p1_send = sems[0]
        p1_recv = sems[1]
        cw_send = sems[2:2 + N_Q]
        cw_recv = sems[6:6 + N_Q]
        ccw_send = sems[10:10 + N_Q]
        ccw_recv = sems[14:14 + N_Q]

        my_x = lax.axis_index("x")
        my_y = lax.axis_index("y")
        my_z = lax.axis_index("z")
        partner = (1 - my_x, my_y, my_z)

        r = jnp.where(my_y == 0, my_z, 7 - my_z)

        def ring_coords(p):
            p = p % N_RING
            py = (p >= 4).astype(my_z.dtype)
            pz = jnp.where(p < 4, p, 7 - p)
            return (my_x, py, pz)

        nxt = ring_coords(r + 1)
        prv = ring_coords(r - 1)

        def qs(k, q):
            return pl.ds((k % N_RING) * rows + q * qrows, qrows)

        my_rows = pl.ds(r * rows, rows)
        cp = pltpu.make_async_copy(x_hbm.at[my_rows], xchunk, copy_sem)
        cp.start()

        barrier_sem = pltpu.get_barrier_semaphore()
        for dev in (partner, nxt, prv):
            pl.semaphore_signal(
                barrier_sem, inc=1, device_id=dev,
                device_id_type=pl.DeviceIdType.MESH,
            )
        pl.semaphore_wait(barrier_sem, 3)

        cp.wait()
        mysend[...] = xchunk[...].astype(jnp.bfloat16)

        sends = []

        def start(d):
            d.start()
            sends.append(d)

        def rcopy(slc, send_sems, recv_sems, idx, dev):
            return pltpu.make_async_remote_copy(
                src_ref=out_ref.at[slc],
                dst_ref=out_ref.at[slc],
                send_sem=send_sems.at[idx],
                recv_sem=recv_sems.at[idx],
                device_id=dev,
                device_id_type=pl.DeviceIdType.MESH,
            )

        p1 = []
        for q in range(N_Q):
            sl = pl.ds(q * qrows, qrows)
            d = pltpu.make_async_remote_copy(
                src_ref=mysend.at[sl],
                dst_ref=p1recv.at[sl],
                send_sem=p1_send.at[q],
                recv_sem=p1_recv.at[q],
                device_id=partner,
                device_id_type=pl.DeviceIdType.MESH,
            )
            start(d)
            p1.append(d)

        for q in range(N_Q):
            p1[q].wait_recv()
            sl = pl.ds(q * qrows, qrows)
            out_ref[qs(r, q), :] = (
                xchunk[sl, :] + p1recv[sl, :].astype(jnp.float32)
            ).astype(jnp.bfloat16)
            start(rcopy(qs(r, q), cw_send[q], cw_recv[q], 0, nxt))
            start(rcopy(qs(r, q), ccw_send[q], ccw_recv[q], 0, prv))

        lanes = (
            [("cw", q) for q in (0, 1)]
            + [("ccw", q) for q in (2, 3)]
            + [("cw", q) for q in (2, 3)]
            + [("ccw", q) for q in (0, 1)]
        )
        for j in range(4):
            for dirn, q in lanes:
                if dirn == "cw":
                    depth, dev = CW_DEPTH[q], nxt
                    ssem, rsem = cw_send[q], cw_recv[q]
                    k = r - 1 - j
                else:
                    depth, dev = CCW_DEPTH[q], prv
                    ssem, rsem = ccw_send[q], ccw_recv[q]
                    k = r + 1 + j
                if j < depth:
                    rcopy(qs(k, q), ssem, rsem, j, dev).wait_recv()
                    if j + 1 < depth:
                        start(rcopy(qs(k, q), ssem, rsem, j + 1, dev))

        for d in sends:
            d.wait_send()

    qsem = pltpu.SemaphoreType.DMA
    return pl.pallas_call(
        body,
        out_shape=jax.ShapeDtypeStruct((m, n), jnp.bfloat16),
        in_specs=[pl.BlockSpec(memory_space=pl.ANY)],
        out_specs=pl.BlockSpec(memory_space=pltpu.VMEM),
        scratch_shapes=[
            pltpu.VMEM((rows, n), jnp.float32),
            pltpu.VMEM((rows, n), jnp.bfloat16),
            pltpu.VMEM((rows, n), jnp.bfloat16),
            qsem,
            qsem((N_Q,)), qsem((N_Q,)),
            *[qsem((CW_DEPTH[q],)) for q in range(N_Q)],
            *[qsem((CW_DEPTH[q],)) for q in range(N_Q)],
            *[qsem((CCW_DEPTH[q],)) for q in range(N_Q)],
            *[qsem((CCW_DEPTH[q],)) for q in range(N_Q)],
        ],
        compiler_params=pltpu.CompilerParams(collective_id=0),
    )(x)


# device time: 52618 ns/iter; 1.1898x vs baseline; 1.0152x over previous
import jax
import jax.numpy as jnp
from jax import lax
from jax.experimental import pallas as pl
from jax.experimental.pallas import tpu as pltpu

N_RING = 8
N_Q = 8
CW_DEPTH = (4, 4, 4, 4, 3, 3, 3, 3)
CCW_DEPTH = (3, 3, 3, 3, 4, 4, 4, 4)


def kernel(x):
    m, n = x.shape
    rows = m // N_RING
    qrows = rows // N_Q

    def body(x_hbm, out_ref, xchunk, mysend, p1recv, copy_sem, *sems):
        p1_send = sems[0]
        p1_recv = sems[1]
        cw_send = sems[2:2 + N_Q]
        cw_recv = sems[2 + N_Q:2 + 2 * N_Q]
        ccw_send = sems[2 + 2 * N_Q:2 + 3 * N_Q]
        ccw_recv = sems[2 + 3 * N_Q:2 + 4 * N_Q]

        my_x = lax.axis_index("x")
        my_y = lax.axis_index("y")
        my_z = lax.axis_index("z")
        partner = (1 - my_x, my_y, my_z)

        r = jnp.where(my_y == 0, my_z, 7 - my_z)

        def ring_coords(p):
            p = p % N_RING
            py = (p >= 4).astype(my_z.dtype)
            pz = jnp.where(p < 4, p, 7 - p)
            return (my_x, py, pz)

        nxt = ring_coords(r + 1)
        prv = ring_coords(r - 1)

        def qs(k, q):
            return pl.ds((k % N_RING) * rows + q * qrows, qrows)

        my_rows = pl.ds(r * rows, rows)
        cp = pltpu.make_async_copy(x_hbm.at[my_rows], xchunk, copy_sem)
        cp.start()

        barrier_sem = pltpu.get_barrier_semaphore()
        for dev in (partner, nxt, prv):
            pl.semaphore_signal(
                barrier_sem, inc=1, device_id=dev,
                device_id_type=pl.DeviceIdType.MESH,
            )
        pl.semaphore_wait(barrier_sem, 3)

        cp.wait()
        mysend[...] = xchunk[...].astype(jnp.bfloat16)

        sends = []

        def start(d):
            d.start()
            sends.append(d)

        def rcopy(slc, send_sems, recv_sems, idx, dev):
            return pltpu.make_async_remote_copy(
                src_ref=out_ref.at[slc],
                dst_ref=out_ref.at[slc],
                send_sem=send_sems.at[idx],
                recv_sem=recv_sems.at[idx],
                device_id=dev,
                device_id_type=pl.DeviceIdType.MESH,
            )

        p1 = []
        for q in range(N_Q):
            sl = pl.ds(q * qrows, qrows)
            d = pltpu.make_async_remote_copy(
                src_ref=mysend.at[sl],
                dst_ref=p1recv.at[sl],
                send_sem=p1_send.at[q],
                recv_sem=p1_recv.at[q],
                device_id=partner,
                device_id_type=pl.DeviceIdType.MESH,
            )
            start(d)
            p1.append(d)

        for q in range(N_Q):
            p1[q].wait_recv()
            sl = pl.ds(q * qrows, qrows)
            out_ref[qs(r, q), :] = (
                xchunk[sl, :] + p1recv[sl, :].astype(jnp.float32)
            ).astype(jnp.bfloat16)
            start(rcopy(qs(r, q), cw_send[q], cw_recv[q], 0, nxt))
            start(rcopy(qs(r, q), ccw_send[q], ccw_recv[q], 0, prv))

        deep_cw = [q for q in range(N_Q) if CW_DEPTH[q] == 4]
        shal_cw = [q for q in range(N_Q) if CW_DEPTH[q] < 4]
        lanes = (
            [("cw", q) for q in deep_cw]
            + [("ccw", q) for q in shal_cw]
            + [("cw", q) for q in shal_cw]
            + [("ccw", q) for q in deep_cw]
        )
        for j in range(4):
            for dirn, q in lanes:
                if dirn == "cw":
                    depth, dev = CW_DEPTH[q], nxt
                    ssem, rsem = cw_send[q], cw_recv[q]
                    k = r - 1 - j
                else:
                    depth, dev = CCW_DEPTH[q], prv
                    ssem, rsem = ccw_send[q], ccw_recv[q]
                    k = r + 1 + j
                if j < depth:
                    rcopy(qs(k, q), ssem, rsem, j, dev).wait_recv()
                    if j + 1 < depth:
                        start(rcopy(qs(k, q), ssem, rsem, j + 1, dev))

        for d in sends:
            d.wait_send()

    qsem = pltpu.SemaphoreType.DMA
    return pl.pallas_call(
        body,
        out_shape=jax.ShapeDtypeStruct((m, n), jnp.bfloat16),
        in_specs=[pl.BlockSpec(memory_space=pl.ANY)],
        out_specs=pl.BlockSpec(memory_space=pltpu.VMEM),
        scratch_shapes=[
            pltpu.VMEM((rows, n), jnp.float32),
            pltpu.VMEM((rows, n), jnp.bfloat16),
            pltpu.VMEM((rows, n), jnp.bfloat16),
            qsem,
            qsem((N_Q,)), qsem((N_Q,)),
            *[qsem((CW_DEPTH[q],)) for q in range(N_Q)],
            *[qsem((CW_DEPTH[q],)) for q in range(N_Q)],
            *[qsem((CCW_DEPTH[q],)) for q in range(N_Q)],
            *[qsem((CCW_DEPTH[q],)) for q in range(N_Q)],
        ],
        compiler_params=pltpu.CompilerParams(collective_id=0),
    )(x)


# device time: 51637 ns/iter; 1.2124x vs baseline; 1.0190x over previous
import jax
import jax.numpy as jnp
from jax import lax
from jax.experimental import pallas as pl
from jax.experimental.pallas import tpu as pltpu

N_RING = 8
N_Q = 4
CW_DEPTH = (4, 4, 2, 1)
CCW_DEPTH = (3, 3, 2, 3)
XF_SEND = {
    ("cw", 0, 2): 0,
    ("cw", 0, 3): 1,
    ("ccw", 0, 2): 2,
    ("cw", 1, 1): 3,
    ("cw", 1, 2): 4,
    ("cw", 1, 3): 5,
}
XF_RECV = [
    (2, -3), (2, -4), (2, +3),
    (3, -2), (3, -3), (3, -4),
]


def kernel(x):
    m, n = x.shape
    rows = m // N_RING
    qrows = rows // N_Q

    def body(x_hbm, out_ref, xchunk, mysend, p1recv, copy_sem, *sems):
        p1_send = sems[0]
        p1_recv = sems[1]
        cw_send = sems[2:2 + N_Q]
        cw_recv = sems[2 + N_Q:2 + 2 * N_Q]
        ccw_send = sems[2 + 2 * N_Q:2 + 3 * N_Q]
        ccw_recv = sems[2 + 3 * N_Q:2 + 4 * N_Q]
        xf_send = sems[2 + 4 * N_Q]
        xf_recv = sems[3 + 4 * N_Q]

        my_x = lax.axis_index("x")
        my_y = lax.axis_index("y")
        my_z = lax.axis_index("z")
        partner = (1 - my_x, my_y, my_z)

        def pq(l):
            return (l + 2 * my_x) % N_Q

        r = jnp.where(my_y == 0, my_z, 7 - my_z)

        def ring_coords(p):
            p = p % N_RING
            py = (p >= 4).astype(my_z.dtype)
            pz = jnp.where(p < 4, p, 7 - p)
            return (my_x, py, pz)

        nxt = ring_coords(r + 1)
        prv = ring_coords(r - 1)

        def qs(k, q):
            return pl.ds((k % N_RING) * rows + q * qrows, qrows)

        my_rows = pl.ds(r * rows, rows)
        cp = pltpu.make_async_copy(x_hbm.at[my_rows], xchunk, copy_sem)
        cp.start()

        barrier_sem = pltpu.get_barrier_semaphore()
        for dev in (partner, nxt, prv):
            pl.semaphore_signal(
                barrier_sem, inc=1, device_id=dev,
                device_id_type=pl.DeviceIdType.MESH,
            )
        pl.semaphore_wait(barrier_sem, 3)

        cp.wait()
        mysend[...] = xchunk[...].astype(jnp.bfloat16)

        sends = []

        def start(d):
            d.start()
            sends.append(d)

        def rcopy(slc, send_sems, recv_sems, idx, dev):
            return pltpu.make_async_remote_copy(
                src_ref=out_ref.at[slc],
                dst_ref=out_ref.at[slc],
                send_sem=send_sems.at[idx],
                recv_sem=recv_sems.at[idx],
                device_id=dev,
                device_id_type=pl.DeviceIdType.MESH,
            )

        p1 = []
        for i in range(N_Q):
            send_q = (i + 2 * (1 - my_x)) % N_Q
            sl = pl.ds(send_q * qrows, qrows)
            d = pltpu.make_async_remote_copy(
                src_ref=mysend.at[sl],
                dst_ref=p1recv.at[sl],
                send_sem=p1_send.at[i],
                recv_sem=p1_recv.at[i],
                device_id=partner,
                device_id_type=pl.DeviceIdType.MESH,
            )
            start(d)
            p1.append(d)

        for l in range(N_Q):
            p1[l].wait_recv()
            q = pq(l)
            sl = pl.ds(q * qrows, qrows)
            out_ref[qs(r, q), :] = (
                xchunk[sl, :] + p1recv[sl, :].astype(jnp.float32)
            ).astype(jnp.bfloat16)
            start(rcopy(qs(r, q), cw_send[l], cw_recv[l], 0, nxt))
            start(rcopy(qs(r, q), ccw_send[l], ccw_recv[l], 0, prv))

        lanes = [
            ("cw", 0), ("cw", 1),
            ("ccw", 0), ("ccw", 1), ("ccw", 3),
            ("cw", 2), ("ccw", 2), ("cw", 3),
        ]
        for j in range(4):
            for dirn, l in lanes:
                if dirn == "cw":
                    depth, dev = CW_DEPTH[l], nxt
                    ssem, rsem = cw_send[l], cw_recv[l]
                    k = r - 1 - j
                else:
                    depth, dev = CCW_DEPTH[l], prv
                    ssem, rsem = ccw_send[l], ccw_recv[l]
                    k = r + 1 + j
                if j < depth:
                    rcopy(qs(k, pq(l)), ssem, rsem, j, dev).wait_recv()
                    if j + 1 < depth:
                        start(rcopy(qs(k, pq(l)), ssem, rsem, j + 1, dev))
                    xi = XF_SEND.get((dirn, l, j))
                    if xi is not None:
                        start(rcopy(qs(k, pq(l)), xf_send, xf_recv, xi, partner))

        for i, (l, off) in enumerate(XF_RECV):
            rcopy(qs(r + off, pq(l)), xf_send, xf_recv, i, partner).wait_recv()

        for d in sends:
            d.wait_send()

    qsem = pltpu.SemaphoreType.DMA
    return pl.pallas_call(
        body,
        out_shape=jax.ShapeDtypeStruct((m, n), jnp.bfloat16),
        in_specs=[pl.BlockSpec(memory_space=pl.ANY)],
        out_specs=pl.BlockSpec(memory_space=pltpu.VMEM),
        scratch_shapes=[
            pltpu.VMEM((rows, n), jnp.float32),
            pltpu.VMEM((rows, n), jnp.bfloat16),
            pltpu.VMEM((rows, n), jnp.bfloat16),
            qsem,
            qsem((N_Q,)), qsem((N_Q,)),
            *[qsem((CW_DEPTH[l],)) for l in range(N_Q)],
            *[qsem((CW_DEPTH[l],)) for l in range(N_Q)],
            *[qsem((CCW_DEPTH[l],)) for l in range(N_Q)],
            *[qsem((CCW_DEPTH[l],)) for l in range(N_Q)],
            qsem((6,)), qsem((6,)),
        ],
        compiler_params=pltpu.CompilerParams(collective_id=0),
    )(x)


# device time: 51634 ns/iter; 1.2125x vs baseline; 1.0001x over previous
import jax
import jax.numpy as jnp
from jax import lax
from jax.experimental import pallas as pl
from jax.experimental.pallas import tpu as pltpu

N_RING = 8
N_Q = 4
CW_DEPTH = (4, 4, 2, 2)
CCW_DEPTH = (2, 2, 4, 4)
N_XF = 4
XF_SEND = {
    ("cw", 0, 2): 0,
    ("cw", 1, 2): 1,
    ("ccw", 2, 2): 2,
    ("ccw", 3, 2): 3,
}
XF_RECV = [
    (2, -3), (3, -3), (0, +3), (1, +3),
]


def kernel(x):
    m, n = x.shape
    rows = m // N_RING
    qrows = rows // N_Q

    def body(x_hbm, out_ref, xchunk, mysend, p1recv, copy_sem, *sems):
        p1_send = sems[0]
        p1_recv = sems[1]
        cw_send = sems[2:2 + N_Q]
        cw_recv = sems[2 + N_Q:2 + 2 * N_Q]
        ccw_send = sems[2 + 2 * N_Q:2 + 3 * N_Q]
        ccw_recv = sems[2 + 3 * N_Q:2 + 4 * N_Q]
        xf_send = sems[2 + 4 * N_Q]
        xf_recv = sems[3 + 4 * N_Q]

        my_x = lax.axis_index("x")
        my_y = lax.axis_index("y")
        my_z = lax.axis_index("z")
        partner = (1 - my_x, my_y, my_z)

        def pq(l):
            return (l + 2 * my_x) % N_Q

        r = jnp.where(my_y == 0, my_z, 7 - my_z)

        def ring_coords(p):
            p = p % N_RING
            py = (p >= 4).astype(my_z.dtype)
            pz = jnp.where(p < 4, p, 7 - p)
            return (my_x, py, pz)

        nxt = ring_coords(r + 1)
        prv = ring_coords(r - 1)

        def qs(k, q):
            return pl.ds((k % N_RING) * rows + q * qrows, qrows)

        my_rows = pl.ds(r * rows, rows)
        cp = pltpu.make_async_copy(x_hbm.at[my_rows], xchunk, copy_sem)
        cp.start()

        barrier_sem = pltpu.get_barrier_semaphore()
        for dev in (partner, nxt, prv):
            pl.semaphore_signal(
                barrier_sem, inc=1, device_id=dev,
                device_id_type=pl.DeviceIdType.MESH,
            )
        pl.semaphore_wait(barrier_sem, 3)

        cp.wait()
        mysend[...] = xchunk[...].astype(jnp.bfloat16)

        sends = []

        def start(d):
            d.start()
            sends.append(d)

        def rcopy(slc, send_sems, recv_sems, idx, dev):
            return pltpu.make_async_remote_copy(
                src_ref=out_ref.at[slc],
                dst_ref=out_ref.at[slc],
                send_sem=send_sems.at[idx],
                recv_sem=recv_sems.at[idx],
                device_id=dev,
                device_id_type=pl.DeviceIdType.MESH,
            )

        p1 = []
        for i in range(N_Q):
            send_q = (i + 2 * (1 - my_x)) % N_Q
            sl = pl.ds(send_q * qrows, qrows)
            d = pltpu.make_async_remote_copy(
                src_ref=mysend.at[sl],
                dst_ref=p1recv.at[sl],
                send_sem=p1_send.at[i],
                recv_sem=p1_recv.at[i],
                device_id=partner,
                device_id_type=pl.DeviceIdType.MESH,
            )
            start(d)
            p1.append(d)

        for l in range(N_Q):
            p1[l].wait_recv()
            q = pq(l)
            sl = pl.ds(q * qrows, qrows)
            out_ref[qs(r, q), :] = (
                xchunk[sl, :] + p1recv[sl, :].astype(jnp.float32)
            ).astype(jnp.bfloat16)
            start(rcopy(qs(r, q), cw_send[l], cw_recv[l], 0, nxt))
            start(rcopy(qs(r, q), ccw_send[l], ccw_recv[l], 0, prv))

        lanes = [
            ("cw", 0), ("cw", 1), ("ccw", 2), ("ccw", 3),
            ("cw", 2), ("cw", 3), ("ccw", 0), ("ccw", 1),
        ]
        for j in range(4):
            for dirn, l in lanes:
                if dirn == "cw":
                    depth, dev = CW_DEPTH[l], nxt
                    ssem, rsem = cw_send[l], cw_recv[l]
                    k = r - 1 - j
                else:
                    depth, dev = CCW_DEPTH[l], prv
                    ssem, rsem = ccw_send[l], ccw_recv[l]
                    k = r + 1 + j
                if j < depth:
                    rcopy(qs(k, pq(l)), ssem, rsem, j, dev).wait_recv()
                    if j + 1 < depth:
                        start(rcopy(qs(k, pq(l)), ssem, rsem, j + 1, dev))
                    xi = XF_SEND.get((dirn, l, j))
                    if xi is not None:
                        start(rcopy(qs(k, pq(l)), xf_send, xf_recv, xi, partner))

        for i, (l, off) in enumerate(XF_RECV):
            rcopy(qs(r + off, pq(l)), xf_send, xf_recv, i, partner).wait_recv()

        for d in sends:
            d.wait_send()

    qsem = pltpu.SemaphoreType.DMA
    return pl.pallas_call(
        body,
        out_shape=jax.ShapeDtypeStruct((m, n), jnp.bfloat16),
        in_specs=[pl.BlockSpec(memory_space=pl.ANY)],
        out_specs=pl.BlockSpec(memory_space=pltpu.VMEM),
        scratch_shapes=[
            pltpu.VMEM((rows, n), jnp.float32),
            pltpu.VMEM((rows, n), jnp.bfloat16),
            pltpu.VMEM((rows, n), jnp.bfloat16),
            qsem,
            qsem((N_Q,)), qsem((N_Q,)),
            *[qsem((CW_DEPTH[l],)) for l in range(N_Q)],
            *[qsem((CW_DEPTH[l],)) for l in range(N_Q)],
            *[qsem((CCW_DEPTH[l],)) for l in range(N_Q)],
            *[qsem((CCW_DEPTH[l],)) for l in range(N_Q)],
            qsem((N_XF,)), qsem((N_XF,)),
        ],
        compiler_params=pltpu.CompilerParams(collective_id=0),
    )(x)
